# baseline (speedup 1.0000x reference)
import functools

import jax
import jax.numpy as jnp
from jax import lax
from jax.experimental import pallas as pl
from jax.experimental.pallas import tpu as pltpu
from jax.experimental.pallas import tpu_sc as plsc

_NC = 2
_NS = 16
_NW = _NC * _NS
_CHUNK = 100


@functools.partial(jax.jit, static_argnames=("B", "L", "D"))
def _pool(x2, table, B, L, D):
    n_chunks = L // _CHUNK
    rows_per_w = B // _NW
    mesh = plsc.VectorSubcoreMesh(core_axis_name="c", subcore_axis_name="s")

    @functools.partial(
        pl.kernel,
        out_type=jax.ShapeDtypeStruct((B, D), jnp.float32),
        mesh=mesh,
        scratch_types=[
            pltpu.VMEM((n_chunks, _CHUNK), jnp.int32),
            pltpu.VMEM((L, D), jnp.float32),
            pltpu.VMEM((D,), jnp.float32),
            pltpu.SemaphoreType.DMA,
        ],
        compiler_params=pltpu.CompilerParams(use_tc_tiling_on_sc=False),
    )
    def body(x_hbm, table_hbm, out_hbm, idx_v, rows_v, pooled_v, sem):
        wid = lax.axis_index("s") * _NC + lax.axis_index("c")
        base = wid * rows_per_w

        def row_body(i, carry):
            row = base + i
            pltpu.sync_copy(x_hbm.at[pl.ds(row * n_chunks, n_chunks)], idx_v)
            cps = []
            for c in range(n_chunks):
                cps.append(
                    pltpu.async_copy(
                        table_hbm.at[idx_v.at[c]],
                        rows_v.at[pl.ds(c * _CHUNK, _CHUNK)],
                        sem,
                    )
                )
            for cp in cps:
                cp.wait()

            zero = jnp.zeros((16,), jnp.float32)
            nv = D // 16

            def acc_body(r, accs):
                return tuple(
                    accs[k] + rows_v[r, pl.ds(k * 16, 16)] for k in range(nv)
                )

            accs = lax.fori_loop(0, L, acc_body, (zero,) * nv)
            scale = jnp.float32(1.0 / L)
            for k in range(nv):
                pooled_v[pl.ds(k * 16, 16)] = accs[k] * scale
            pltpu.sync_copy(pooled_v, out_hbm.at[row])
            return carry

        lax.fori_loop(0, rows_per_w, row_body, 0)

    return body(x2, table)


def _mm_body(p_ref, w_ref, b_ref, o_ref):
    o_ref[...] = (
        jnp.dot(p_ref[...], w_ref[...], preferred_element_type=jnp.float32)
        + b_ref[...]
    )


def kernel(x, table, W, b):
    B, L = x.shape
    V, D = table.shape
    x2 = x.astype(jnp.int32).reshape(B * (L // _CHUNK), _CHUNK)
    pooled = _pool(x2, table, B=B, L=L, D=D)
    out = pl.pallas_call(
        _mm_body,
        out_shape=jax.ShapeDtypeStruct((B, W.shape[1]), jnp.float32),
    )(pooled, W, b.reshape(1, -1))
    return out
